# Initial kernel scaffold; baseline (speedup 1.0000x reference)
import jax
import jax.numpy as jnp
from jax.experimental import pallas as pl


def kernel(student_features, teacher_features, teacher_codes, codebook):
    raise NotImplementedError("write your pallas kernel here")


# speedup vs baseline: 4.7360x; 4.7360x over previous
import jax
import jax.numpy as jnp
from jax.experimental import pallas as pl
from jax.experimental.pallas import tpu as pltpu

_B, _T, _D, _K = 4, 1024, 32, 8192
_NUM_NEG = 16
_TEMP = 0.1
_MARGIN = 0.2
_FEAT_W = 1.0
_TRIP_W = 1.0
_CONTR_W = 0.5

_TILE = 256
_N = _B * _T
_NB = _N // _TILE
_TPB = _T // _TILE


def _loss_kernel(s_ref, t_ref, nt_ref, codes_ref, cb_ref, out_ref, acc_ref):
    g = pl.program_id(0)

    @pl.when(g == 0)
    def _init():
        acc_ref[0] = 0.0
        acc_ref[1] = 0.0
        acc_ref[2] = 0.0

    s = s_ref[...]
    t = t_ref[...]
    nt = nt_ref[...]
    codes = codes_ref[0, 0, :]

    diff = s - t
    sq = jnp.sum(diff * diff, axis=1)
    feat = jnp.sum(sq) * (1.0 / _D)
    pos_dist = jnp.sqrt(sq)
    ndiff = s - nt
    neg_dist = jnp.sqrt(jnp.sum(ndiff * ndiff, axis=1))
    trip = jnp.sum(jnp.maximum(pos_dist - neg_dist + _MARGIN, 0.0))

    cb = cb_ref[...]
    scores = jax.lax.dot_general(
        s, cb, (((1,), (1,)), ((), ())), preferred_element_type=jnp.float32
    )
    x2 = jnp.sum(s * s, axis=1, keepdims=True)
    c2 = jnp.sum(cb * cb, axis=1)[None, :]
    d2 = jnp.maximum(x2 + c2 - 2.0 * scores, 0.0)

    col = jax.lax.broadcasted_iota(jnp.int32, (_TILE, _K), 1)
    is_pos = col == codes[:, None]

    inv_nx = 1.0 / jnp.maximum(jnp.sqrt(x2), 1e-12)
    inv_nc = 1.0 / jnp.maximum(jnp.sqrt(c2), 1e-12)
    cosn = scores * inv_nx * inv_nc * (1.0 / _TEMP)

    inf = jnp.float32(jnp.inf)
    d2m = jnp.where(is_pos, inf, d2)
    pos_sim = jnp.sum(jnp.where(is_pos, cosn, 0.0), axis=1)

    nege = jnp.zeros((_TILE,), jnp.float32)
    for _ in range(_NUM_NEG):
        m = jnp.min(d2m, axis=1, keepdims=True)
        idx = jnp.min(jnp.where(d2m == m, col, _K), axis=1, keepdims=True)
        sel = col == idx
        nege = nege + jnp.exp(jnp.sum(jnp.where(sel, cosn, 0.0), axis=1) - 10.0)
        d2m = jnp.where(sel, inf, d2m)

    ce = jnp.sum(jnp.log(jnp.exp(pos_sim - 10.0) + nege) + 10.0 - pos_sim)

    acc_ref[0] += feat
    acc_ref[1] += trip
    acc_ref[2] += ce

    @pl.when(g == _NB - 1)
    def _fin():
        total = (
            _FEAT_W * acc_ref[0] + _TRIP_W * acc_ref[1] + _CONTR_W * acc_ref[2]
        ) * (1.0 / _N)
        out_ref[...] = jnp.full((1, 1), total, dtype=jnp.float32)


def kernel(student_features, teacher_features, teacher_codes, codebook):
    s = student_features.reshape(_N, _D)
    t = teacher_features.reshape(_N, _D)
    codes = teacher_codes.reshape(_NB, 1, _TILE).astype(jnp.int32)
    out = pl.pallas_call(
        _loss_kernel,
        grid=(_NB,),
        in_specs=[
            pl.BlockSpec((_TILE, _D), lambda g: (g, 0)),
            pl.BlockSpec((_TILE, _D), lambda g: (g, 0)),
            pl.BlockSpec(
                (_TILE, _D),
                lambda g: (((g // _TPB - 1) % _B) * _TPB + g % _TPB, 0),
            ),
            pl.BlockSpec((1, 1, _TILE), lambda g: (g, 0, 0)),
            pl.BlockSpec((_K, _D), lambda g: (0, 0)),
        ],
        out_specs=pl.BlockSpec((1, 1), lambda g: (0, 0)),
        out_shape=jax.ShapeDtypeStruct((1, 1), jnp.float32),
        scratch_shapes=[pltpu.SMEM((3,), jnp.float32)],
        compiler_params=pltpu.CompilerParams(
            dimension_semantics=("arbitrary",),
        ),
    )(s, t, t, codes, codebook)
    return out[0, 0]


# speedup vs baseline: 5.8354x; 1.2321x over previous
import jax
import jax.numpy as jnp
from jax.experimental import pallas as pl
from jax.experimental.pallas import tpu as pltpu

_B, _T, _D, _K = 4, 1024, 32, 8192
_NUM_NEG = 16
_TEMP = 0.1
_MARGIN = 0.2
_FEAT_W = 1.0
_TRIP_W = 1.0
_CONTR_W = 0.5

_TILE = 128
_N = _B * _T
_NB = _N // _TILE
_TPB = _T // _TILE
_NJ = _NUM_NEG + 2


def _loss_kernel(
    s_ref, t_ref, nt_ref, codes_ref, cb_ref, out_ref,
    acc_ref, keym_ref, ecos_ref, m_ref, nege_ref, epos_ref,
):
    g = pl.program_id(0)
    j = pl.program_id(1)
    inf = jnp.float32(jnp.inf)

    @pl.when(jnp.logical_and(g == 0, j == 0))
    def _init_acc():
        acc_ref[0] = 0.0
        acc_ref[1] = 0.0
        acc_ref[2] = 0.0

    @pl.when(j == 0)
    def _prologue():
        s = s_ref[...]
        t = t_ref[...]
        nt = nt_ref[...]
        codes = codes_ref[0, 0, :]

        diff = s - t
        sq = jnp.sum(diff * diff, axis=1)
        pos_dist = jnp.sqrt(sq)
        ndiff = s - nt
        neg_dist = jnp.sqrt(jnp.sum(ndiff * ndiff, axis=1))
        acc_ref[0] += jnp.sum(sq) * (1.0 / _D)
        acc_ref[1] += jnp.sum(jnp.maximum(pos_dist - neg_dist + _MARGIN, 0.0))

        cb = cb_ref[...]
        scores = jax.lax.dot_general(
            s, cb, (((1,), (1,)), ((), ())), preferred_element_type=jnp.float32
        )
        x2 = jnp.sum(s * s, axis=1, keepdims=True)
        c2 = jnp.sum(cb * cb, axis=1)[None, :]

        col = jax.lax.broadcasted_iota(jnp.int32, (_TILE, _K), 1)
        is_pos = col == codes[:, None]

        keym_ref[...] = jnp.where(is_pos, inf, x2 + c2 - 2.0 * scores)

        inv_nx = 1.0 / jnp.maximum(jnp.sqrt(x2), 1e-12)
        inv_nc = 1.0 / jnp.maximum(jnp.sqrt(c2), 1e-12)
        ecos = jnp.exp(scores * inv_nx * inv_nc * (1.0 / _TEMP) - 10.0)
        ecos_ref[...] = ecos

        epos_ref[...] = jnp.sum(
            jnp.where(is_pos, ecos, 0.0), axis=1, keepdims=True
        )
        nege_ref[...] = jnp.zeros((_TILE, 1), jnp.float32)
        m_ref[...] = jnp.min(keym_ref[...], axis=1, keepdims=True)

    @pl.when(jnp.logical_and(j >= 1, j <= _NUM_NEG))
    def _extract():
        k = keym_ref[...]
        e = ecos_ref[...]
        sel = k == m_ref[...]
        nege_ref[...] += jnp.sum(jnp.where(sel, e, 0.0), axis=1, keepdims=True)
        k2 = jnp.where(sel, inf, k)
        keym_ref[...] = k2
        m_ref[...] = jnp.min(k2, axis=1, keepdims=True)

    @pl.when(j == _NJ - 1)
    def _epilogue():
        epos = epos_ref[...]
        nege = nege_ref[...]
        acc_ref[2] += jnp.sum(jnp.log(epos + nege) - jnp.log(epos))

        @pl.when(g == _NB - 1)
        def _finalize():
            total = (
                _FEAT_W * acc_ref[0]
                + _TRIP_W * acc_ref[1]
                + _CONTR_W * acc_ref[2]
            ) * (1.0 / _N)
            out_ref[...] = jnp.full((1, 1), total, dtype=jnp.float32)


def kernel(student_features, teacher_features, teacher_codes, codebook):
    s = student_features.reshape(_N, _D)
    t = teacher_features.reshape(_N, _D)
    codes = teacher_codes.reshape(_NB, 1, _TILE).astype(jnp.int32)
    out = pl.pallas_call(
        _loss_kernel,
        grid=(_NB, _NJ),
        in_specs=[
            pl.BlockSpec((_TILE, _D), lambda g, j: (g, 0)),
            pl.BlockSpec((_TILE, _D), lambda g, j: (g, 0)),
            pl.BlockSpec(
                (_TILE, _D),
                lambda g, j: (((g // _TPB - 1) % _B) * _TPB + g % _TPB, 0),
            ),
            pl.BlockSpec((1, 1, _TILE), lambda g, j: (g, 0, 0)),
            pl.BlockSpec((_K, _D), lambda g, j: (0, 0)),
        ],
        out_specs=pl.BlockSpec((1, 1), lambda g, j: (0, 0)),
        out_shape=jax.ShapeDtypeStruct((1, 1), jnp.float32),
        scratch_shapes=[
            pltpu.SMEM((3,), jnp.float32),
            pltpu.VMEM((_TILE, _K), jnp.float32),
            pltpu.VMEM((_TILE, _K), jnp.float32),
            pltpu.VMEM((_TILE, 1), jnp.float32),
            pltpu.VMEM((_TILE, 1), jnp.float32),
            pltpu.VMEM((_TILE, 1), jnp.float32),
        ],
        compiler_params=pltpu.CompilerParams(
            dimension_semantics=("arbitrary", "arbitrary"),
        ),
    )(s, t, t, codes, codebook)
    return out[0, 0]


# speedup vs baseline: 5.9490x; 1.0195x over previous
import jax
import jax.numpy as jnp
from jax.experimental import pallas as pl
from jax.experimental.pallas import tpu as pltpu

_B, _T, _D, _K = 4, 1024, 32, 8192
_NUM_NEG = 16
_TEMP = 0.1
_MARGIN = 0.2
_FEAT_W = 1.0
_TRIP_W = 1.0
_CONTR_W = 0.5

_TILE = 128
_N = _B * _T
_NB = _N // _TILE
_TPB = _T // _TILE
_NJ = _NUM_NEG + 2


def _loss_kernel(
    s_ref, t_ref, nt_ref, codes_ref, cb_ref, out_ref,
    acc_ref, keym_ref, c2_ref, x2_ref, inx_ref, m_ref, nege_ref, epos_ref,
):
    g = pl.program_id(0)
    j = pl.program_id(1)
    inf = jnp.float32(jnp.inf)

    @pl.when(jnp.logical_and(g == 0, j == 0))
    def _init_acc():
        acc_ref[0] = 0.0
        acc_ref[1] = 0.0
        acc_ref[2] = 0.0

    @pl.when(j == 0)
    def _prologue():
        s = s_ref[...]
        t = t_ref[...]
        nt = nt_ref[...]
        codes = codes_ref[0, 0, :]

        diff = s - t
        sq = jnp.sum(diff * diff, axis=1)
        pos_dist = jnp.sqrt(sq)
        ndiff = s - nt
        neg_dist = jnp.sqrt(jnp.sum(ndiff * ndiff, axis=1))
        acc_ref[0] += jnp.sum(sq) * (1.0 / _D)
        acc_ref[1] += jnp.sum(jnp.maximum(pos_dist - neg_dist + _MARGIN, 0.0))

        cb = cb_ref[...]
        scores = jax.lax.dot_general(
            s, cb, (((1,), (1,)), ((), ())), preferred_element_type=jnp.float32
        )
        x2 = jnp.sum(s * s, axis=1, keepdims=True)
        c2 = jnp.sum(cb * cb, axis=1)[None, :]

        col = jax.lax.broadcasted_iota(jnp.int32, (_TILE, _K), 1)
        is_pos = col == codes[:, None]

        keym = jnp.where(is_pos, inf, x2 + c2 - 2.0 * scores)
        keym_ref[...] = keym

        inv_nx = 1.0 / jnp.maximum(jnp.sqrt(x2), 1e-12)

        spos = jnp.sum(jnp.where(is_pos, scores, 0.0), axis=1, keepdims=True)
        c2pos = jnp.sum(is_pos.astype(jnp.float32) * c2, axis=1, keepdims=True)
        inv_np = 1.0 / jnp.maximum(jnp.sqrt(c2pos), 1e-12)
        pos_sim = spos * inv_nx * inv_np * (1.0 / _TEMP)
        epos_ref[...] = jnp.exp(pos_sim - 10.0)

        c2_ref[...] = c2
        x2_ref[...] = x2
        inx_ref[...] = inv_nx
        nege_ref[...] = jnp.zeros((_TILE, 1), jnp.float32)
        m_ref[...] = jnp.min(keym, axis=1, keepdims=True)

    @pl.when(jnp.logical_and(j >= 1, j <= _NUM_NEG))
    def _extract():
        k = keym_ref[...]
        m_prev = m_ref[...]
        c2 = c2_ref[...]

        sel = (k == m_prev).astype(jnp.float32)
        c2s = jnp.sum(sel * c2, axis=1, keepdims=True)

        sdot = (x2_ref[...] + c2s - m_prev) * 0.5
        inv_nc = 1.0 / jnp.maximum(jnp.sqrt(c2s), 1e-12)
        coslog = sdot * inx_ref[...] * inv_nc * (1.0 / _TEMP)
        nege_ref[...] += jnp.exp(coslog - 10.0)

        m_ref[...] = jnp.min(
            jnp.where(k > m_prev, k, inf), axis=1, keepdims=True
        )

    @pl.when(j == _NJ - 1)
    def _epilogue():
        epos = epos_ref[...]
        nege = nege_ref[...]
        acc_ref[2] += jnp.sum(jnp.log(epos + nege) - jnp.log(epos))

        @pl.when(g == _NB - 1)
        def _finalize():
            total = (
                _FEAT_W * acc_ref[0]
                + _TRIP_W * acc_ref[1]
                + _CONTR_W * acc_ref[2]
            ) * (1.0 / _N)
            out_ref[...] = jnp.full((1, 1), total, dtype=jnp.float32)


def kernel(student_features, teacher_features, teacher_codes, codebook):
    s = student_features.reshape(_N, _D)
    t = teacher_features.reshape(_N, _D)
    codes = teacher_codes.reshape(_NB, 1, _TILE).astype(jnp.int32)
    out = pl.pallas_call(
        _loss_kernel,
        grid=(_NB, _NJ),
        in_specs=[
            pl.BlockSpec((_TILE, _D), lambda g, j: (g, 0)),
            pl.BlockSpec((_TILE, _D), lambda g, j: (g, 0)),
            pl.BlockSpec(
                (_TILE, _D),
                lambda g, j: (((g // _TPB - 1) % _B) * _TPB + g % _TPB, 0),
            ),
            pl.BlockSpec((1, 1, _TILE), lambda g, j: (g, 0, 0)),
            pl.BlockSpec((_K, _D), lambda g, j: (0, 0)),
        ],
        out_specs=pl.BlockSpec((1, 1), lambda g, j: (0, 0)),
        out_shape=jax.ShapeDtypeStruct((1, 1), jnp.float32),
        scratch_shapes=[
            pltpu.SMEM((3,), jnp.float32),
            pltpu.VMEM((_TILE, _K), jnp.float32),
            pltpu.VMEM((1, _K), jnp.float32),
            pltpu.VMEM((_TILE, 1), jnp.float32),
            pltpu.VMEM((_TILE, 1), jnp.float32),
            pltpu.VMEM((_TILE, 1), jnp.float32),
            pltpu.VMEM((_TILE, 1), jnp.float32),
            pltpu.VMEM((_TILE, 1), jnp.float32),
        ],
        compiler_params=pltpu.CompilerParams(
            dimension_semantics=("arbitrary", "arbitrary"),
        ),
    )(s, t, t, codes, codebook)
    return out[0, 0]


# speedup vs baseline: 7.4814x; 1.2576x over previous
import jax
import jax.numpy as jnp
from jax.experimental import pallas as pl
from jax.experimental.pallas import tpu as pltpu

_B, _T, _D, _K = 4, 1024, 32, 8192
_NUM_NEG = 16
_TEMP = 0.1
_MARGIN = 0.2
_FEAT_W = 1.0
_TRIP_W = 1.0
_CONTR_W = 0.5

_TILE = 128
_N = _B * _T
_NB = _N // _TILE
_TPB = _T // _TILE
_NJ = _NUM_NEG + 2

_QBITS = 13
_QMASK = (1 << _QBITS) - 1
_IMAX = 2147483647


def _loss_kernel(
    s_ref, t_ref, nt_ref, codes_ref, cb_ref, out_ref,
    acc_ref, key_ref, m_ref, nege_ref, epos_ref,
):
    g = pl.program_id(0)
    j = pl.program_id(1)

    @pl.when(jnp.logical_and(g == 0, j == 0))
    def _init_acc():
        acc_ref[0] = 0.0
        acc_ref[1] = 0.0
        acc_ref[2] = 0.0

    @pl.when(j == 0)
    def _prologue():
        s = s_ref[...]
        t = t_ref[...]
        nt = nt_ref[...]
        codes = codes_ref[0, 0, :]

        diff = s - t
        sq = jnp.sum(diff * diff, axis=1)
        pos_dist = jnp.sqrt(sq)
        ndiff = s - nt
        neg_dist = jnp.sqrt(jnp.sum(ndiff * ndiff, axis=1))
        acc_ref[0] += jnp.sum(sq) * (1.0 / _D)
        acc_ref[1] += jnp.sum(jnp.maximum(pos_dist - neg_dist + _MARGIN, 0.0))

        cb = cb_ref[...]
        scores = jax.lax.dot_general(
            s, cb, (((1,), (1,)), ((), ())), preferred_element_type=jnp.float32
        )
        x2 = jnp.sum(s * s, axis=1, keepdims=True)
        c2 = jnp.sum(cb * cb, axis=1)[None, :]

        col = jax.lax.broadcasted_iota(jnp.int32, (_TILE, _K), 1)
        is_pos = col == codes[:, None]

        inv_nx = 1.0 / jnp.maximum(jnp.sqrt(x2), 1e-12)
        inv_nc = 1.0 / jnp.maximum(jnp.sqrt(c2), 1e-12)
        coslog = scores * inv_nx * inv_nc * (1.0 / _TEMP)

        pos_sim = jnp.sum(jnp.where(is_pos, coslog, 0.0), axis=1, keepdims=True)
        epos_ref[...] = jnp.exp(pos_sim - 10.0)

        d2 = jnp.maximum(x2 + c2 - 2.0 * scores, 0.0)
        q = jnp.clip(
            (coslog + 10.0) * (_QMASK / 20.0) + 0.5, 0.0, float(_QMASK)
        ).astype(jnp.int32)
        ikey = (jax.lax.bitcast_convert_type(d2, jnp.int32) & (~_QMASK)) | q
        ikey = jnp.where(is_pos, _IMAX, ikey)
        key_ref[...] = ikey
        m_ref[...] = jnp.min(ikey, axis=1, keepdims=True)

        nege_ref[...] = jnp.zeros((_TILE, 1), jnp.float32)

    @pl.when(jnp.logical_and(j >= 1, j <= _NUM_NEG))
    def _extract():
        m_prev = m_ref[...]
        qv = (m_prev & _QMASK).astype(jnp.float32)
        coslog = qv * (20.0 / _QMASK) - 10.0
        nege_ref[...] += jnp.exp(coslog - 10.0)

        k = key_ref[...]
        m_ref[...] = jnp.min(
            jnp.where(k > m_prev, k, _IMAX), axis=1, keepdims=True
        )

    @pl.when(j == _NJ - 1)
    def _epilogue():
        epos = epos_ref[...]
        nege = nege_ref[...]
        acc_ref[2] += jnp.sum(jnp.log(epos + nege) - jnp.log(epos))

        @pl.when(g == _NB - 1)
        def _finalize():
            total = (
                _FEAT_W * acc_ref[0]
                + _TRIP_W * acc_ref[1]
                + _CONTR_W * acc_ref[2]
            ) * (1.0 / _N)
            out_ref[...] = jnp.full((1, 1), total, dtype=jnp.float32)


def kernel(student_features, teacher_features, teacher_codes, codebook):
    s = student_features.reshape(_N, _D)
    t = teacher_features.reshape(_N, _D)
    codes = teacher_codes.reshape(_NB, 1, _TILE).astype(jnp.int32)
    out = pl.pallas_call(
        _loss_kernel,
        grid=(_NB, _NJ),
        in_specs=[
            pl.BlockSpec((_TILE, _D), lambda g, j: (g, 0)),
            pl.BlockSpec((_TILE, _D), lambda g, j: (g, 0)),
            pl.BlockSpec(
                (_TILE, _D),
                lambda g, j: (((g // _TPB - 1) % _B) * _TPB + g % _TPB, 0),
            ),
            pl.BlockSpec((1, 1, _TILE), lambda g, j: (g, 0, 0)),
            pl.BlockSpec((_K, _D), lambda g, j: (0, 0)),
        ],
        out_specs=pl.BlockSpec((1, 1), lambda g, j: (0, 0)),
        out_shape=jax.ShapeDtypeStruct((1, 1), jnp.float32),
        scratch_shapes=[
            pltpu.SMEM((3,), jnp.float32),
            pltpu.VMEM((_TILE, _K), jnp.int32),
            pltpu.VMEM((_TILE, 1), jnp.int32),
            pltpu.VMEM((_TILE, 1), jnp.float32),
            pltpu.VMEM((_TILE, 1), jnp.float32),
        ],
        compiler_params=pltpu.CompilerParams(
            dimension_semantics=("arbitrary", "arbitrary"),
        ),
    )(s, t, t, codes, codebook)
    return out[0, 0]


# speedup vs baseline: 8.3283x; 1.1132x over previous
import jax
import jax.numpy as jnp
from jax.experimental import pallas as pl
from jax.experimental.pallas import tpu as pltpu

_B, _T, _D, _K = 4, 1024, 32, 8192
_NUM_NEG = 16
_TEMP = 0.1
_MARGIN = 0.2
_FEAT_W = 1.0
_TRIP_W = 1.0
_CONTR_W = 0.5

_TILE = 128
_N = _B * _T
_NB = _N // _TILE
_TPB = _T // _TILE
_NJ = _NUM_NEG + 2

_QBITS = 13
_QMASK = (1 << _QBITS) - 1
_IMAX = 2147483647
_QSCALE = _QMASK / 20.0
_QOFF = 0.5 * _QMASK + 0.5


def _loss_kernel(
    s_ref, t_ref, nt_ref, codes_ref, cb_ref, out_ref,
    ft_ref, key_ref, m_ref, nege_ref, epos_ref,
):
    g = pl.program_id(0)
    j = pl.program_id(1)

    @pl.when(j == 0)
    def _prologue():
        s = s_ref[...]
        t = t_ref[...]
        nt = nt_ref[...]
        codes = codes_ref[0, 0, :]

        diff = s - t
        sq = jnp.sum(diff * diff, axis=1)
        pos_dist = jnp.sqrt(sq)
        ndiff = s - nt
        neg_dist = jnp.sqrt(jnp.sum(ndiff * ndiff, axis=1))
        ft_ref[0] = _FEAT_W * jnp.sum(sq) * (1.0 / _D) + _TRIP_W * jnp.sum(
            jnp.maximum(pos_dist - neg_dist + _MARGIN, 0.0)
        )

        cb = cb_ref[...]
        scores = jax.lax.dot_general(
            s, cb, (((1,), (1,)), ((), ())), preferred_element_type=jnp.float32
        )
        sq_cb = cb * cb
        c2 = jax.lax.dot_general(
            jnp.ones((1, _D), jnp.float32), sq_cb,
            (((1,), (1,)), ((), ())), preferred_element_type=jnp.float32,
        )
        x2 = jnp.sum(s * s, axis=1, keepdims=True)

        col = jax.lax.broadcasted_iota(jnp.int32, (_TILE, _K), 1)
        is_pos = col == codes[:, None]

        inv_nx = jax.lax.rsqrt(jnp.maximum(x2, 1e-24))
        a_row = inv_nx * (_QSCALE / _TEMP)
        inv_nc = jax.lax.rsqrt(jnp.maximum(c2, 1e-24))
        qf = scores * a_row * inv_nc + _QOFF
        q = jnp.clip(qf, 0.0, float(_QMASK)).astype(jnp.int32)

        pos_qf = jnp.sum(jnp.where(is_pos, qf, 0.0), axis=1, keepdims=True)
        pos_sim = (pos_qf - _QOFF) * (1.0 / _QSCALE)
        epos_ref[...] = jnp.exp(pos_sim - 10.0)

        d2 = jnp.maximum(x2 + c2 - 2.0 * scores, 0.0)
        ikey = (jax.lax.bitcast_convert_type(d2, jnp.int32) & (~_QMASK)) | q
        ikey = jnp.where(is_pos, _IMAX, ikey)
        key_ref[...] = ikey
        m_ref[...] = jnp.min(ikey, axis=1, keepdims=True)

        nege_ref[...] = jnp.zeros((_TILE, 1), jnp.float32)

    @pl.when(jnp.logical_and(j >= 1, j <= _NUM_NEG))
    def _extract():
        m_prev = m_ref[...]
        qv = (m_prev & _QMASK).astype(jnp.float32)
        coslog = qv * (1.0 / _QSCALE) - 10.0
        nege_ref[...] += jnp.exp(coslog - 10.0)

        k = key_ref[...]
        m_ref[...] = jnp.min(
            jnp.where(k > m_prev, k, _IMAX), axis=1, keepdims=True
        )

    @pl.when(j == _NJ - 1)
    def _epilogue():
        epos = epos_ref[...]
        nege = nege_ref[...]
        ce = jnp.sum(jnp.log(epos + nege) - jnp.log(epos))
        out_ref[...] = jnp.full(
            (1, 1, 1), (ft_ref[0] + _CONTR_W * ce) * (1.0 / _N),
            dtype=jnp.float32,
        )


def _reduce_kernel(p_ref, out_ref):
    out_ref[...] = jnp.full((1, 1), jnp.sum(p_ref[...]), dtype=jnp.float32)


def kernel(student_features, teacher_features, teacher_codes, codebook):
    s = student_features.reshape(_N, _D)
    t = teacher_features.reshape(_N, _D)
    codes = teacher_codes.reshape(_NB, 1, _TILE).astype(jnp.int32)
    partials = pl.pallas_call(
        _loss_kernel,
        grid=(_NB, _NJ),
        in_specs=[
            pl.BlockSpec((_TILE, _D), lambda g, j: (g, 0)),
            pl.BlockSpec((_TILE, _D), lambda g, j: (g, 0)),
            pl.BlockSpec(
                (_TILE, _D),
                lambda g, j: (((g // _TPB - 1) % _B) * _TPB + g % _TPB, 0),
            ),
            pl.BlockSpec((1, 1, _TILE), lambda g, j: (g, 0, 0)),
            pl.BlockSpec((_K, _D), lambda g, j: (0, 0)),
        ],
        out_specs=pl.BlockSpec((1, 1, 1), lambda g, j: (g, 0, 0)),
        out_shape=jax.ShapeDtypeStruct((_NB, 1, 1), jnp.float32),
        scratch_shapes=[
            pltpu.SMEM((1,), jnp.float32),
            pltpu.VMEM((_TILE, _K), jnp.int32),
            pltpu.VMEM((_TILE, 1), jnp.int32),
            pltpu.VMEM((_TILE, 1), jnp.float32),
            pltpu.VMEM((_TILE, 1), jnp.float32),
        ],
        compiler_params=pltpu.CompilerParams(
            dimension_semantics=("parallel", "arbitrary"),
        ),
    )(s, t, t, codes, codebook)
    out = pl.pallas_call(
        _reduce_kernel,
        out_shape=jax.ShapeDtypeStruct((1, 1), jnp.float32),
    )(partials.reshape(_NB, 1))
    return out[0, 0]


# speedup vs baseline: 9.6627x; 1.1602x over previous
import jax
import jax.numpy as jnp
from jax.experimental import pallas as pl
from jax.experimental.pallas import tpu as pltpu

_B, _T, _D, _K = 4, 1024, 32, 8192
_NUM_NEG = 16
_TEMP = 0.1
_MARGIN = 0.2
_FEAT_W = 1.0
_TRIP_W = 1.0
_CONTR_W = 0.5

_TILE = 128
_N = _B * _T
_NB = _N // _TILE
_TPB = _T // _TILE
_NPASS = _NUM_NEG // 2
_NJ = _NPASS + 2
_CHUNK = 256

_QBITS = 13
_QMASK = (1 << _QBITS) - 1
_IMAX = 2147483647
_QSCALE = _QMASK / 20.0
_QOFF = 0.5 * _QMASK + 0.5


def _loss_kernel(
    s_ref, t_ref, nt_ref, codes_ref, cb_ref, out_ref,
    ft_ref, key_ref, m_ref, nege_ref, epos_ref,
):
    g = pl.program_id(0)
    j = pl.program_id(1)

    @pl.when(j == 0)
    def _prologue():
        s = s_ref[...]
        t = t_ref[...]
        nt = nt_ref[...]
        codes = codes_ref[0, 0, :]

        diff = s - t
        sq = jnp.sum(diff * diff, axis=1)
        pos_dist = jnp.sqrt(sq)
        ndiff = s - nt
        neg_dist = jnp.sqrt(jnp.sum(ndiff * ndiff, axis=1))
        ft_ref[0] = _FEAT_W * jnp.sum(sq) * (1.0 / _D) + _TRIP_W * jnp.sum(
            jnp.maximum(pos_dist - neg_dist + _MARGIN, 0.0)
        )

        cb = cb_ref[...]
        scores_n2 = jax.lax.dot_general(
            s * (-2.0), cb, (((1,), (1,)), ((), ())),
            preferred_element_type=jnp.float32,
        )
        sq_cb = cb * cb
        c2 = jax.lax.dot_general(
            jnp.ones((1, _D), jnp.float32), sq_cb,
            (((1,), (1,)), ((), ())), preferred_element_type=jnp.float32,
        )
        x2 = jnp.sum(s * s, axis=1, keepdims=True)

        col = jax.lax.broadcasted_iota(jnp.int32, (_TILE, _K), 1)
        is_pos = col == codes[:, None]

        inv_nx = jax.lax.rsqrt(jnp.maximum(x2, 1e-24))
        a_row = inv_nx * (-0.5 * _QSCALE / _TEMP)
        inv_nc = jax.lax.rsqrt(jnp.maximum(c2, 1e-24))
        qf = scores_n2 * a_row * inv_nc + _QOFF
        q = qf.astype(jnp.int32)

        pos_qf = jnp.sum(jnp.where(is_pos, qf, 0.0), axis=1, keepdims=True)
        pos_sim = (pos_qf - _QOFF) * (1.0 / _QSCALE)
        epos_ref[...] = jnp.exp(pos_sim - 10.0)

        d2 = jnp.maximum(x2 + c2 + scores_n2, 0.0)
        ikey = (jax.lax.bitcast_convert_type(d2, jnp.int32) & (~_QMASK)) | q
        ikey = jnp.where(is_pos, _IMAX, ikey)
        key_ref[...] = ikey
        m_ref[...] = jnp.full((_TILE, 1), -1, jnp.int32)

        nege_ref[...] = jnp.zeros((_TILE, 1), jnp.float32)

    @pl.when(jnp.logical_and(j >= 1, j <= _NPASS))
    def _extract():
        m_prev = m_ref[...]
        a = jnp.full((_TILE, _CHUNK), _IMAX, jnp.int32)
        b = jnp.full((_TILE, _CHUNK), _IMAX, jnp.int32)
        for c in range(_K // _CHUNK):
            x = key_ref[:, c * _CHUNK:(c + 1) * _CHUNK]
            xm = jnp.where(x > m_prev, x, _IMAX)
            t = jnp.maximum(a, xm)
            a = jnp.minimum(a, xm)
            b = jnp.minimum(b, t)
        m1 = jnp.min(a, axis=1, keepdims=True)
        m2 = jnp.minimum(
            jnp.min(jnp.where(a > m1, a, _IMAX), axis=1, keepdims=True),
            jnp.min(b, axis=1, keepdims=True),
        )
        e1 = jnp.exp((m1 & _QMASK).astype(jnp.float32) * (1.0 / _QSCALE) - 20.0)
        e2 = jnp.exp((m2 & _QMASK).astype(jnp.float32) * (1.0 / _QSCALE) - 20.0)
        nege_ref[...] += e1 + e2
        m_ref[...] = m2

    @pl.when(j == _NJ - 1)
    def _epilogue():
        epos = epos_ref[...]
        nege = nege_ref[...]
        ce = jnp.sum(jnp.log(epos + nege) - jnp.log(epos))
        out_ref[...] = jnp.full(
            (1, 1, 1), (ft_ref[0] + _CONTR_W * ce) * (1.0 / _N),
            dtype=jnp.float32,
        )


def _reduce_kernel(p_ref, out_ref):
    out_ref[...] = jnp.full((1, 1), jnp.sum(p_ref[...]), dtype=jnp.float32)


def kernel(student_features, teacher_features, teacher_codes, codebook):
    s = student_features.reshape(_N, _D)
    t = teacher_features.reshape(_N, _D)
    codes = teacher_codes.reshape(_NB, 1, _TILE).astype(jnp.int32)
    partials = pl.pallas_call(
        _loss_kernel,
        grid=(_NB, _NJ),
        in_specs=[
            pl.BlockSpec((_TILE, _D), lambda g, j: (g, 0)),
            pl.BlockSpec((_TILE, _D), lambda g, j: (g, 0)),
            pl.BlockSpec(
                (_TILE, _D),
                lambda g, j: (((g // _TPB - 1) % _B) * _TPB + g % _TPB, 0),
            ),
            pl.BlockSpec((1, 1, _TILE), lambda g, j: (g, 0, 0)),
            pl.BlockSpec((_K, _D), lambda g, j: (0, 0)),
        ],
        out_specs=pl.BlockSpec((1, 1, 1), lambda g, j: (g, 0, 0)),
        out_shape=jax.ShapeDtypeStruct((_NB, 1, 1), jnp.float32),
        scratch_shapes=[
            pltpu.SMEM((1,), jnp.float32),
            pltpu.VMEM((_TILE, _K), jnp.int32),
            pltpu.VMEM((_TILE, 1), jnp.int32),
            pltpu.VMEM((_TILE, 1), jnp.float32),
            pltpu.VMEM((_TILE, 1), jnp.float32),
        ],
        compiler_params=pltpu.CompilerParams(
            dimension_semantics=("parallel", "arbitrary"),
        ),
    )(s, t, t, codes, codebook)
    out = pl.pallas_call(
        _reduce_kernel,
        out_shape=jax.ShapeDtypeStruct((1, 1), jnp.float32),
    )(partials.reshape(_NB, 1))
    return out[0, 0]


# speedup vs baseline: 10.2903x; 1.0649x over previous
import jax
import jax.numpy as jnp
from jax.experimental import pallas as pl
from jax.experimental.pallas import tpu as pltpu

_B, _T, _D, _K = 4, 1024, 32, 8192
_NUM_NEG = 16
_TEMP = 0.1
_MARGIN = 0.2
_FEAT_W = 1.0
_TRIP_W = 1.0
_CONTR_W = 0.5

_TILE = 256
_N = _B * _T
_NB = _N // _TILE
_TPB = _T // _TILE
_NPASS = _NUM_NEG // 2
_NJ = _NPASS + 2
_CHUNK = 256

_QBITS = 13
_QMASK = (1 << _QBITS) - 1
_IMAX = 2147483647
_QSCALE = _QMASK / 20.0
_QOFF = 0.5 * _QMASK + 0.5


def _loss_kernel(
    s_ref, t_ref, nt_ref, codes_ref, cb_ref, out_ref,
    ft_ref, key_ref, m_ref, nege_ref, epos_ref,
):
    g = pl.program_id(0)
    j = pl.program_id(1)

    @pl.when(j == 0)
    def _prologue():
        s = s_ref[...]
        t = t_ref[...]
        nt = nt_ref[...]
        codes = codes_ref[0, 0, :]

        diff = s - t
        sq = jnp.sum(diff * diff, axis=1)
        pos_dist = jnp.sqrt(sq)
        ndiff = s - nt
        neg_dist = jnp.sqrt(jnp.sum(ndiff * ndiff, axis=1))
        ft_ref[0] = _FEAT_W * jnp.sum(sq) * (1.0 / _D) + _TRIP_W * jnp.sum(
            jnp.maximum(pos_dist - neg_dist + _MARGIN, 0.0)
        )

        cb = cb_ref[...]
        scores_n2 = jax.lax.dot_general(
            s * (-2.0), cb, (((1,), (1,)), ((), ())),
            preferred_element_type=jnp.float32,
        )
        sq_cb = cb * cb
        c2 = jax.lax.dot_general(
            jnp.ones((1, _D), jnp.float32), sq_cb,
            (((1,), (1,)), ((), ())), preferred_element_type=jnp.float32,
        )
        x2 = jnp.sum(s * s, axis=1, keepdims=True)

        col = jax.lax.broadcasted_iota(jnp.int32, (_TILE, _K), 1)
        is_pos = col == codes[:, None]

        inv_nx = jax.lax.rsqrt(jnp.maximum(x2, 1e-24))
        a_row = inv_nx * (-0.5 * _QSCALE / _TEMP)
        inv_nc = jax.lax.rsqrt(jnp.maximum(c2, 1e-24))
        qf = scores_n2 * a_row * inv_nc + _QOFF
        q = qf.astype(jnp.int32)

        pos_qf = jnp.sum(jnp.where(is_pos, qf, 0.0), axis=1, keepdims=True)
        pos_sim = (pos_qf - _QOFF) * (1.0 / _QSCALE)
        epos_ref[...] = jnp.exp(pos_sim - 10.0)

        d2 = jnp.maximum(x2 + c2 + scores_n2, 0.0)
        ikey = (jax.lax.bitcast_convert_type(d2, jnp.int32) & (~_QMASK)) | q
        ikey = jnp.where(is_pos, _IMAX, ikey)
        key_ref[...] = ikey
        m_ref[...] = jnp.full((_TILE, 1), -1, jnp.int32)

        nege_ref[...] = jnp.zeros((_TILE, 1), jnp.float32)

    @pl.when(jnp.logical_and(j >= 1, j <= _NPASS))
    def _extract():
        m_prev = m_ref[...]
        a = jnp.full((_TILE, _CHUNK), _IMAX, jnp.int32)
        b = jnp.full((_TILE, _CHUNK), _IMAX, jnp.int32)
        for c in range(_K // _CHUNK):
            x = key_ref[:, c * _CHUNK:(c + 1) * _CHUNK]
            xm = jnp.where(x > m_prev, x, _IMAX)
            t = jnp.maximum(a, xm)
            a = jnp.minimum(a, xm)
            b = jnp.minimum(b, t)
        m1 = jnp.min(a, axis=1, keepdims=True)
        m2 = jnp.minimum(
            jnp.min(jnp.where(a > m1, a, _IMAX), axis=1, keepdims=True),
            jnp.min(b, axis=1, keepdims=True),
        )
        e1 = jnp.exp((m1 & _QMASK).astype(jnp.float32) * (1.0 / _QSCALE) - 20.0)
        e2 = jnp.exp((m2 & _QMASK).astype(jnp.float32) * (1.0 / _QSCALE) - 20.0)
        nege_ref[...] += e1 + e2
        m_ref[...] = m2

    @pl.when(j == _NJ - 1)
    def _epilogue():
        epos = epos_ref[...]
        nege = nege_ref[...]
        ce = jnp.sum(jnp.log(epos + nege) - jnp.log(epos))
        out_ref[...] = jnp.full(
            (1, 1, 1), (ft_ref[0] + _CONTR_W * ce) * (1.0 / _N),
            dtype=jnp.float32,
        )


def _reduce_kernel(p_ref, out_ref):
    out_ref[...] = jnp.full((1, 1), jnp.sum(p_ref[...]), dtype=jnp.float32)


def kernel(student_features, teacher_features, teacher_codes, codebook):
    s = student_features.reshape(_N, _D)
    t = teacher_features.reshape(_N, _D)
    codes = teacher_codes.reshape(_NB, 1, _TILE).astype(jnp.int32)
    partials = pl.pallas_call(
        _loss_kernel,
        grid=(_NB, _NJ),
        in_specs=[
            pl.BlockSpec((_TILE, _D), lambda g, j: (g, 0)),
            pl.BlockSpec((_TILE, _D), lambda g, j: (g, 0)),
            pl.BlockSpec(
                (_TILE, _D),
                lambda g, j: (((g // _TPB - 1) % _B) * _TPB + g % _TPB, 0),
            ),
            pl.BlockSpec((1, 1, _TILE), lambda g, j: (g, 0, 0)),
            pl.BlockSpec((_K, _D), lambda g, j: (0, 0)),
        ],
        out_specs=pl.BlockSpec((1, 1, 1), lambda g, j: (g, 0, 0)),
        out_shape=jax.ShapeDtypeStruct((_NB, 1, 1), jnp.float32),
        scratch_shapes=[
            pltpu.SMEM((1,), jnp.float32),
            pltpu.VMEM((_TILE, _K), jnp.int32),
            pltpu.VMEM((_TILE, 1), jnp.int32),
            pltpu.VMEM((_TILE, 1), jnp.float32),
            pltpu.VMEM((_TILE, 1), jnp.float32),
        ],
        compiler_params=pltpu.CompilerParams(
            dimension_semantics=("parallel", "arbitrary"),
        ),
    )(s, t, t, codes, codebook)
    out = pl.pallas_call(
        _reduce_kernel,
        out_shape=jax.ShapeDtypeStruct((1, 1), jnp.float32),
    )(partials.reshape(_NB, 1))
    return out[0, 0]


# speedup vs baseline: 10.4746x; 1.0179x over previous
import jax
import jax.numpy as jnp
from jax.experimental import pallas as pl
from jax.experimental.pallas import tpu as pltpu

_B, _T, _D, _K = 4, 1024, 32, 8192
_NUM_NEG = 16
_TEMP = 0.1
_MARGIN = 0.2
_FEAT_W = 1.0
_TRIP_W = 1.0
_CONTR_W = 0.5

_TILE = 256
_N = _B * _T
_NB = _N // _TILE
_TPB = _T // _TILE
_NPASS = _NUM_NEG // 4
_NJ = _NPASS + 2
_CHUNK = 256

_QBITS = 13
_QMASK = (1 << _QBITS) - 1
_IMAX = 2147483647
_QSCALE = _QMASK / 20.0
_QOFF = 0.5 * _QMASK + 0.5


def _loss_kernel(
    s_ref, t_ref, nt_ref, codes_ref, cb_ref, out_ref,
    ft_ref, key_ref, m_ref, nege_ref, epos_ref,
):
    g = pl.program_id(0)
    j = pl.program_id(1)

    @pl.when(j == 0)
    def _prologue():
        s = s_ref[...]
        t = t_ref[...]
        nt = nt_ref[...]
        codes = codes_ref[0, 0, :]

        diff = s - t
        sq = jnp.sum(diff * diff, axis=1)
        pos_dist = jnp.sqrt(sq)
        ndiff = s - nt
        neg_dist = jnp.sqrt(jnp.sum(ndiff * ndiff, axis=1))
        ft_ref[0] = _FEAT_W * jnp.sum(sq) * (1.0 / _D) + _TRIP_W * jnp.sum(
            jnp.maximum(pos_dist - neg_dist + _MARGIN, 0.0)
        )

        cb = cb_ref[...]
        scores_n2 = jax.lax.dot_general(
            s * (-2.0), cb, (((1,), (1,)), ((), ())),
            preferred_element_type=jnp.float32,
        )
        sq_cb = cb * cb
        c2 = jax.lax.dot_general(
            jnp.ones((1, _D), jnp.float32), sq_cb,
            (((1,), (1,)), ((), ())), preferred_element_type=jnp.float32,
        )
        x2 = jnp.sum(s * s, axis=1, keepdims=True)

        col = jax.lax.broadcasted_iota(jnp.int32, (_TILE, _K), 1)
        is_pos = col == codes[:, None]

        inv_nx = jax.lax.rsqrt(jnp.maximum(x2, 1e-24))
        a_row = inv_nx * (-0.5 * _QSCALE / _TEMP)
        inv_nc = jax.lax.rsqrt(jnp.maximum(c2, 1e-24))
        qf = scores_n2 * a_row * inv_nc + _QOFF
        q = qf.astype(jnp.int32)

        pos_qf = jnp.sum(jnp.where(is_pos, qf, 0.0), axis=1, keepdims=True)
        pos_sim = (pos_qf - _QOFF) * (1.0 / _QSCALE)
        epos_ref[...] = jnp.exp(pos_sim - 10.0)

        d2 = jnp.maximum(x2 + c2 + scores_n2, 0.0)
        ikey = (jax.lax.bitcast_convert_type(d2, jnp.int32) & (~_QMASK)) | q
        ikey = jnp.where(is_pos, _IMAX, ikey)
        key_ref[...] = ikey
        m_ref[...] = jnp.full((_TILE, 1), -1, jnp.int32)

        nege_ref[...] = jnp.zeros((_TILE, 1), jnp.float32)

    @pl.when(jnp.logical_and(j >= 1, j <= _NPASS))
    def _extract():
        m_prev = m_ref[...]
        a = jnp.full((_TILE, _CHUNK), _IMAX, jnp.int32)
        b = jnp.full((_TILE, _CHUNK), _IMAX, jnp.int32)
        cc = jnp.full((_TILE, _CHUNK), _IMAX, jnp.int32)
        dd = jnp.full((_TILE, _CHUNK), _IMAX, jnp.int32)
        for c in range(_K // _CHUNK):
            x = key_ref[:, c * _CHUNK:(c + 1) * _CHUNK]
            xm = jnp.where(x > m_prev, x, _IMAX)
            t1 = jnp.maximum(a, xm)
            a = jnp.minimum(a, xm)
            t2 = jnp.maximum(b, t1)
            b = jnp.minimum(b, t1)
            t3 = jnp.maximum(cc, t2)
            cc = jnp.minimum(cc, t2)
            dd = jnp.minimum(dd, t3)
        cat = jnp.concatenate([a, b, cc, dd], axis=1)
        m1 = jnp.min(cat, axis=1, keepdims=True)
        m2 = jnp.min(jnp.where(cat > m1, cat, _IMAX), axis=1, keepdims=True)
        m3 = jnp.min(jnp.where(cat > m2, cat, _IMAX), axis=1, keepdims=True)
        m4 = jnp.min(jnp.where(cat > m3, cat, _IMAX), axis=1, keepdims=True)
        qsum = (
            jnp.exp((m1 & _QMASK).astype(jnp.float32) * (1.0 / _QSCALE) - 20.0)
            + jnp.exp((m2 & _QMASK).astype(jnp.float32) * (1.0 / _QSCALE) - 20.0)
            + jnp.exp((m3 & _QMASK).astype(jnp.float32) * (1.0 / _QSCALE) - 20.0)
            + jnp.exp((m4 & _QMASK).astype(jnp.float32) * (1.0 / _QSCALE) - 20.0)
        )
        nege_ref[...] += qsum
        m_ref[...] = m4

    @pl.when(j == _NJ - 1)
    def _epilogue():
        epos = epos_ref[...]
        nege = nege_ref[...]
        ce = jnp.sum(jnp.log(epos + nege) - jnp.log(epos))
        out_ref[...] = jnp.full(
            (1, 1, 1), (ft_ref[0] + _CONTR_W * ce) * (1.0 / _N),
            dtype=jnp.float32,
        )


def _reduce_kernel(p_ref, out_ref):
    out_ref[...] = jnp.full((1, 1), jnp.sum(p_ref[...]), dtype=jnp.float32)


def kernel(student_features, teacher_features, teacher_codes, codebook):
    s = student_features.reshape(_N, _D)
    t = teacher_features.reshape(_N, _D)
    codes = teacher_codes.reshape(_NB, 1, _TILE).astype(jnp.int32)
    partials = pl.pallas_call(
        _loss_kernel,
        grid=(_NB, _NJ),
        in_specs=[
            pl.BlockSpec((_TILE, _D), lambda g, j: (g, 0)),
            pl.BlockSpec((_TILE, _D), lambda g, j: (g, 0)),
            pl.BlockSpec(
                (_TILE, _D),
                lambda g, j: (((g // _TPB - 1) % _B) * _TPB + g % _TPB, 0),
            ),
            pl.BlockSpec((1, 1, _TILE), lambda g, j: (g, 0, 0)),
            pl.BlockSpec((_K, _D), lambda g, j: (0, 0)),
        ],
        out_specs=pl.BlockSpec((1, 1, 1), lambda g, j: (g, 0, 0)),
        out_shape=jax.ShapeDtypeStruct((_NB, 1, 1), jnp.float32),
        scratch_shapes=[
            pltpu.SMEM((1,), jnp.float32),
            pltpu.VMEM((_TILE, _K), jnp.int32),
            pltpu.VMEM((_TILE, 1), jnp.int32),
            pltpu.VMEM((_TILE, 1), jnp.float32),
            pltpu.VMEM((_TILE, 1), jnp.float32),
        ],
        compiler_params=pltpu.CompilerParams(
            dimension_semantics=("parallel", "arbitrary"),
        ),
    )(s, t, t, codes, codebook)
    out = pl.pallas_call(
        _reduce_kernel,
        out_shape=jax.ShapeDtypeStruct((1, 1), jnp.float32),
    )(partials.reshape(_NB, 1))
    return out[0, 0]
